# baseline (speedup 1.0000x reference)
import jax
import jax.numpy as jnp
from jax import lax
from jax.experimental import pallas as pl
from jax.experimental.pallas import tpu as pltpu
from jax.experimental.pallas import tpu_sc as plsc

NC = 2
NS = 16
NW = NC * NS
LANE = 16

ROW = 128
JR = 4
PLANE = 2 * JR * 8 * ROW


TILES_PER_BLOCK = 16


def _transpose_body(in_ref, out_ref):
    x = in_ref[...]
    eye = jnp.eye(48, dtype=x.dtype)
    for m in range(TILES_PER_BLOCK):
        xm = x[:, ROW * m:ROW * (m + 1)]
        y = lax.dot_general(xm, eye, (((0,), (0,)), ((), ())),
                            precision=lax.Precision.HIGHEST,
                            preferred_element_type=jnp.float32)
        top, bot = y[:64], y[64:]
        out_ref[64 * m:64 * (m + 1), :] = jnp.concatenate(
            [top, bot[:, :16], bot, top[:, :16]], axis=1)


def _build_table(attributes):
    bs, f, _, D = attributes.shape
    M = TILES_PER_BLOCK
    nj = (f + ROW * M - 1) // (ROW * M)
    fp = nj * M * ROW
    a = 3 * D
    at = jnp.transpose(attributes, (0, 2, 3, 1)).reshape(bs * a, f)
    packed = pl.pallas_call(
        _transpose_body,
        grid=(bs, nj),
        in_specs=[pl.BlockSpec((a, ROW * M), lambda b, j: (b, j))],
        out_specs=pl.BlockSpec((64 * M, ROW),
                               lambda b, j, nj=nj: (b * nj + j, 0)),
        out_shape=jax.ShapeDtypeStruct((bs * nj * 64 * M, ROW), jnp.float32),
    )(at)
    return packed.reshape(bs * fp, 64), fp


def _face_rows(p2f, f, fp):
    fpad = p2f + (fp - f) * (p2f // f)
    t, l = fpad // ROW, fpad % ROW
    return t * ROW + 2 * (l % 64) + l // 64


def _body(idx_hbm, bary_hbm, table_hbm, out_hbm,
          idx_v0, idx_v1, bary_v0, bary_v1, rows_v0, rows_v1,
          out_v0, out_v1,
          in_s0, in_s1, g_s0, g_s1, o_s0, o_s1):
    idx_v = (idx_v0, idx_v1)
    bary_v = (bary_v0, bary_v1)
    rows_v = (rows_v0, rows_v1)
    out_v = (out_v0, out_v1)
    in_s = (in_s0, in_s1)
    g_s = (g_s0, g_s1)
    o_s = (o_s0, o_s1)

    wid = lax.axis_index("s") * NC + lax.axis_index("c")
    n_bh = idx_hbm.shape[0] // JR
    per_w = n_bh // NW
    half = per_w // 2
    base = wid * per_w

    d16 = lax.iota(jnp.int32, LANE)
    vpos = (d16 // 8) * (JR * 8 * ROW) + (d16 % 8) * ROW

    def start_idx(g, s):
        bh = base + g
        pltpu.async_copy(idx_hbm.at[pl.ds(bh * JR, JR)], idx_v[s], in_s[s])

    def start_bary(g, s):
        bh = base + g
        pltpu.async_copy(bary_hbm.at[pl.ds(bh * 3 * 512, 3 * 512)],
                         bary_v[s], in_s[s])

    def start_in(g, s):
        start_idx(g, s)
        start_bary(g, s)

    def wait_in(g, s):
        bh = base + g
        pltpu.make_async_copy(idx_hbm.at[pl.ds(bh * JR, JR)], idx_v[s],
                              in_s[s]).wait()
        pltpu.make_async_copy(bary_hbm.at[pl.ds(bh * 3 * 512, 3 * 512)],
                              bary_v[s], in_s[s]).wait()

    def fire_gathers(s):
        for j in range(JR):
            pltpu.async_copy(table_hbm.at[idx_v[s].at[j]], rows_v[s].at[j],
                             g_s[s])

    def wait_gathers(s):
        for j in range(JR):
            pltpu.make_async_copy(table_hbm.at[idx_v[s].at[j]],
                                  rows_v[s].at[j], g_s[s]).wait()

    def compute(s):
        for j in range(JR):
            def grp(n, c, j=j):
                i0 = 16 * n
                b0w = bary_v[s][pl.ds(j * ROW + i0, LANE)]
                b1w = bary_v[s][pl.ds(512 + j * ROW + i0, LANE)]
                b2w = bary_v[s][pl.ds(1024 + j * ROW + i0, LANE)]
                for p in range(LANE):
                    i = i0 + p
                    r0 = rows_v[s][j, i, pl.ds(0, LANE)]
                    r1 = rows_v[s][j, i, pl.ds(LANE, LANE)]
                    r2 = rows_v[s][j, i, pl.ds(2 * LANE, LANE)]
                    acc = b0w[p] * r0 + b1w[p] * r1 + b2w[p] * r2
                    plsc.store_scatter(out_v[s], [vpos + (j * 8 * ROW + i)],
                                       acc)
                return c
            lax.fori_loop(0, ROW // LANE, grp, 0)

    def start_out(g, s):
        bh = base + g
        pltpu.async_copy(out_v[s], out_hbm.at[pl.ds(bh * PLANE, PLANE)],
                         o_s[s])

    def wait_out(g, s):
        bh = base + g
        pltpu.make_async_copy(out_v[s], out_hbm.at[pl.ds(bh * PLANE, PLANE)],
                              o_s[s]).wait()

    start_in(0, 0)
    start_in(1, 1)
    wait_in(0, 0)
    fire_gathers(0)

    def step(t, carry):
        for b in range(2):
            g = 2 * t + b
            if b == 0:
                wait_in(g + 1, 1 - b)
                fire_gathers(1 - b)
            else:
                @pl.when(t < half - 1)
                def _():
                    wait_in(g + 1, 1 - b)
                    fire_gathers(1 - b)

            wait_gathers(b)

            @pl.when(t >= 1)
            def _():
                wait_out(g - 2, b)

            @pl.when(t < half - 1)
            def _():
                start_idx(g + 2, b)

            compute(b)

            @pl.when(t < half - 1)
            def _():
                start_bary(g + 2, b)

            start_out(g, b)
        return carry

    lax.fori_loop(0, half, step, 0)
    wait_out(per_w - 2, 0)
    wait_out(per_w - 1, 1)


def kernel(pix_to_face, bary_coords, attributes):
    bs, f, _, D = attributes.shape
    B, H, W, K = pix_to_face.shape
    N = B * H * W
    n_rows = N // ROW

    table, fp = _build_table(attributes)
    idx = _face_rows(pix_to_face.astype(jnp.int32), f, fp).reshape(n_rows, ROW)
    bary = jnp.transpose(bary_coords, (0, 1, 4, 3, 2)).reshape(N * 3)

    mesh = plsc.VectorSubcoreMesh(core_axis_name="c", subcore_axis_name="s",
                                  num_cores=NC, num_subcores=NS)
    fn = pl.kernel(
        _body,
        out_type=jax.ShapeDtypeStruct((N * D,), jnp.float32),
        mesh=mesh,
        scratch_types=[
            pltpu.VMEM((JR, ROW), jnp.int32),
            pltpu.VMEM((JR, ROW), jnp.int32),
            pltpu.VMEM((3 * 512,), jnp.float32),
            pltpu.VMEM((3 * 512,), jnp.float32),
            pltpu.VMEM((JR, ROW, 64), jnp.float32),
            pltpu.VMEM((JR, ROW, 64), jnp.float32),
            pltpu.VMEM((PLANE,), jnp.float32),
            pltpu.VMEM((PLANE,), jnp.float32),
            pltpu.SemaphoreType.DMA,
            pltpu.SemaphoreType.DMA,
            pltpu.SemaphoreType.DMA,
            pltpu.SemaphoreType.DMA,
            pltpu.SemaphoreType.DMA,
            pltpu.SemaphoreType.DMA,
        ],
        compiler_params=pltpu.CompilerParams(use_tc_tiling_on_sc=False,
                                             needs_layout_passes=False),
    )
    out = fn(idx, bary, table)
    out = out.reshape(B, H, 2, JR, 8, ROW).transpose(0, 1, 3, 5, 2, 4)
    return out.reshape(B, H, W, D)


# speedup vs baseline: 1.1513x; 1.1513x over previous
import jax
import jax.numpy as jnp
from jax import lax
from jax.experimental import pallas as pl
from jax.experimental.pallas import tpu as pltpu
from jax.experimental.pallas import tpu_sc as plsc

NC = 2
NS = 16
NW = NC * NS
LANE = 16

ROW = 128
JR = 4
PLANE = 2 * JR * 8 * ROW


TILES_PER_BLOCK = 32


def _transpose_body(in_ref, out_ref):
    x = in_ref[...]
    y = x.T
    for m in range(TILES_PER_BLOCK):
        ym = y[ROW * m:ROW * (m + 1)]
        out_ref[64 * m:64 * (m + 1), 0:48] = ym[:64]
        out_ref[64 * m:64 * (m + 1), 64:112] = ym[64:]


def _build_table(attributes):
    bs, f, _, D = attributes.shape
    M = TILES_PER_BLOCK
    nj = (f + ROW * M - 1) // (ROW * M)
    fp = nj * M * ROW
    a = 3 * D
    at = jnp.transpose(attributes, (0, 2, 3, 1)).reshape(bs * a, f)
    packed = pl.pallas_call(
        _transpose_body,
        grid=(bs, nj),
        in_specs=[pl.BlockSpec((a, ROW * M), lambda b, j: (b, j))],
        out_specs=pl.BlockSpec((64 * M, ROW),
                               lambda b, j, nj=nj: (b * nj + j, 0)),
        out_shape=jax.ShapeDtypeStruct((bs * nj * 64 * M, ROW), jnp.float32),
    )(at)
    return packed.reshape(bs * fp, 64), fp


def _face_rows(p2f, f, fp):
    fpad = p2f + (fp - f) * (p2f // f)
    t, l = fpad // ROW, fpad % ROW
    return t * ROW + 2 * (l % 64) + l // 64


def _body(idx_hbm, bary_hbm, table_hbm, out_hbm,
          idx_v0, idx_v1, bary_v0, bary_v1, rows_v0, rows_v1,
          out_v0, out_v1,
          in_s0, in_s1, g_s0, g_s1, o_s0, o_s1):
    idx_v = (idx_v0, idx_v1)
    bary_v = (bary_v0, bary_v1)
    rows_v = (rows_v0, rows_v1)
    out_v = (out_v0, out_v1)
    in_s = (in_s0, in_s1)
    g_s = (g_s0, g_s1)
    o_s = (o_s0, o_s1)

    wid = lax.axis_index("s") * NC + lax.axis_index("c")
    n_bh = idx_hbm.shape[0] // JR
    per_w = n_bh // NW
    half = per_w // 2
    base = wid * per_w

    d16 = lax.iota(jnp.int32, LANE)
    vpos = (d16 // 8) * (JR * 8 * ROW) + (d16 % 8) * ROW

    def start_idx(g, s):
        bh = base + g
        pltpu.async_copy(idx_hbm.at[pl.ds(bh * JR, JR)], idx_v[s], in_s[s])

    def start_bary(g, s):
        bh = base + g
        pltpu.async_copy(bary_hbm.at[pl.ds(bh * 3 * 512, 3 * 512)],
                         bary_v[s], in_s[s])

    def start_in(g, s):
        start_idx(g, s)
        start_bary(g, s)

    def wait_in(g, s):
        bh = base + g
        pltpu.make_async_copy(idx_hbm.at[pl.ds(bh * JR, JR)], idx_v[s],
                              in_s[s]).wait()
        pltpu.make_async_copy(bary_hbm.at[pl.ds(bh * 3 * 512, 3 * 512)],
                              bary_v[s], in_s[s]).wait()

    def fire_gathers(s):
        for j in range(JR):
            pltpu.async_copy(table_hbm.at[idx_v[s].at[j]], rows_v[s].at[j],
                             g_s[s])

    def wait_gathers(s):
        for j in range(JR):
            pltpu.make_async_copy(table_hbm.at[idx_v[s].at[j]],
                                  rows_v[s].at[j], g_s[s]).wait()

    def compute(s):
        for j in range(JR):
            def grp(n, c, j=j):
                i0 = 16 * n
                b0w = bary_v[s][pl.ds(j * ROW + i0, LANE)]
                b1w = bary_v[s][pl.ds(512 + j * ROW + i0, LANE)]
                b2w = bary_v[s][pl.ds(1024 + j * ROW + i0, LANE)]
                for p in range(LANE):
                    i = i0 + p
                    r0 = rows_v[s][j, i, pl.ds(0, LANE)]
                    r1 = rows_v[s][j, i, pl.ds(LANE, LANE)]
                    r2 = rows_v[s][j, i, pl.ds(2 * LANE, LANE)]
                    acc = b0w[p] * r0 + b1w[p] * r1 + b2w[p] * r2
                    plsc.store_scatter(out_v[s], [vpos + (j * 8 * ROW + i)],
                                       acc)
                return c
            lax.fori_loop(0, ROW // LANE, grp, 0)

    def start_out(g, s):
        bh = base + g
        pltpu.async_copy(out_v[s], out_hbm.at[pl.ds(bh * PLANE, PLANE)],
                         o_s[s])

    def wait_out(g, s):
        bh = base + g
        pltpu.make_async_copy(out_v[s], out_hbm.at[pl.ds(bh * PLANE, PLANE)],
                              o_s[s]).wait()

    start_in(0, 0)
    start_in(1, 1)
    wait_in(0, 0)
    fire_gathers(0)

    def step(t, carry):
        for b in range(2):
            g = 2 * t + b
            if b == 0:
                wait_in(g + 1, 1 - b)
                fire_gathers(1 - b)
            else:
                @pl.when(t < half - 1)
                def _():
                    wait_in(g + 1, 1 - b)
                    fire_gathers(1 - b)

            wait_gathers(b)

            @pl.when(t >= 1)
            def _():
                wait_out(g - 2, b)

            @pl.when(t < half - 1)
            def _():
                start_idx(g + 2, b)

            compute(b)

            @pl.when(t < half - 1)
            def _():
                start_bary(g + 2, b)

            start_out(g, b)
        return carry

    lax.fori_loop(0, half, step, 0)
    wait_out(per_w - 2, 0)
    wait_out(per_w - 1, 1)


def kernel(pix_to_face, bary_coords, attributes):
    bs, f, _, D = attributes.shape
    B, H, W, K = pix_to_face.shape
    N = B * H * W
    n_rows = N // ROW

    table, fp = _build_table(attributes)
    idx = _face_rows(pix_to_face.astype(jnp.int32), f, fp).reshape(n_rows, ROW)
    bary = jnp.transpose(bary_coords, (0, 1, 4, 3, 2)).reshape(N * 3)

    mesh = plsc.VectorSubcoreMesh(core_axis_name="c", subcore_axis_name="s",
                                  num_cores=NC, num_subcores=NS)
    fn = pl.kernel(
        _body,
        out_type=jax.ShapeDtypeStruct((N * D,), jnp.float32),
        mesh=mesh,
        scratch_types=[
            pltpu.VMEM((JR, ROW), jnp.int32),
            pltpu.VMEM((JR, ROW), jnp.int32),
            pltpu.VMEM((3 * 512,), jnp.float32),
            pltpu.VMEM((3 * 512,), jnp.float32),
            pltpu.VMEM((JR, ROW, 64), jnp.float32),
            pltpu.VMEM((JR, ROW, 64), jnp.float32),
            pltpu.VMEM((PLANE,), jnp.float32),
            pltpu.VMEM((PLANE,), jnp.float32),
            pltpu.SemaphoreType.DMA,
            pltpu.SemaphoreType.DMA,
            pltpu.SemaphoreType.DMA,
            pltpu.SemaphoreType.DMA,
            pltpu.SemaphoreType.DMA,
            pltpu.SemaphoreType.DMA,
        ],
        compiler_params=pltpu.CompilerParams(use_tc_tiling_on_sc=False,
                                             needs_layout_passes=False),
    )
    out = fn(idx, bary, table)
    out = out.reshape(B, H, 2, JR, 8, ROW).transpose(0, 1, 3, 5, 2, 4)
    return out.reshape(B, H, W, D)
